# baseline (speedup 1.0000x reference)
import functools

import jax
import jax.numpy as jnp
from jax import lax
from jax.experimental import pallas as pl
from jax.experimental.pallas import tpu as pltpu
from jax.experimental.pallas import tpu_sc as plsc

NC = 2
NS = 16
L = 16
K = 128
HB = 128
NB = 3


def _sc_spmm(xT2, rows3, cols3, vals3, bias, n_out, n_batch):
    rpt = n_out // NS
    mesh = plsc.VectorSubcoreMesh(
        core_axis_name="c", subcore_axis_name="s", num_cores=NC,
        num_subcores=NS)

    @functools.partial(
        pl.kernel,
        out_type=jax.ShapeDtypeStruct((NC, n_out, HB), jnp.float32),
        mesh=mesh,
        scratch_types=[
            pltpu.VMEM((NB, K, HB), jnp.float32),
            pltpu.VMEM((n_batch, K), jnp.int32),
            pltpu.VMEM((n_batch, K), jnp.int32),
            pltpu.VMEM((n_batch, K), jnp.float32),
            pltpu.VMEM((rpt,), jnp.float32),
            pltpu.VMEM_SHARED((n_out, HB), jnp.float32),
        ] + [pltpu.SemaphoreType.DMA] * (2 * NB),
    )
    def k(xT2_h, rows_h, cols_h, vals_h, bias_h, out_h,
          xring, rows_v, cols_v, vals_v, bias_v, ysh, *sems):
        cid = lax.axis_index("c")
        sid = lax.axis_index("s")
        gsems = sems[:NB]
        ssems = sems[NB:]
        xbufs = [xring.at[p] for p in range(NB)]

        pltpu.sync_copy(rows_h.at[sid], rows_v)
        pltpu.sync_copy(cols_h.at[sid], cols_v)
        pltpu.sync_copy(vals_h.at[sid], vals_v)
        coff = jnp.zeros((L,), jnp.int32) + cid * (xT2.shape[0] // NC)

        def shift_row(b, _):
            crow = cols_v.at[b]
            for l in range(K // L):
                sl = pl.ds(l * L, L)
                crow[sl] = crow[sl] + coff
            return 0
        lax.fori_loop(0, n_batch, shift_row, 0)

        xb0 = xbufs[0]
        pltpu.sync_copy(bias_h.at[pl.ds(sid * rpt, rpt)], bias_v)
        for h in range(rpt // K):
            def init_group(g, _):
                bv = bias_v[pl.ds(h * K + g * L, L)]
                for jj in range(L):
                    rowvec = jnp.zeros((L,), jnp.float32) + bv[jj]
                    row = xb0.at[g * L + jj]
                    for l in range(HB // L):
                        row[pl.ds(l * L, L)] = rowvec
                return 0
            lax.fori_loop(0, K // L, init_group, 0)
            pltpu.sync_copy(xb0, ysh.at[pl.ds(sid * rpt + h * K, K)])
        plsc.subcore_barrier()

        def start_g(b, p):
            pltpu.async_copy(xT2_h.at[cols_v.at[b]], xbufs[p], gsems[p])

        def wait_g(b, p):
            pltpu.make_async_copy(
                xT2_h.at[cols_v.at[b]], xbufs[p], gsems[p]).wait()

        def start_s(b, p):
            pltpu.async_copy(xbufs[p], ysh.at[rows_v.at[b]], ssems[p],
                             add=True)

        def wait_s(b, p):
            pltpu.make_async_copy(
                xbufs[p], ysh.at[rows_v.at[b]], ssems[p]).wait()

        def scale(b, p):
            xbuf = xbufs[p]
            vrow = vals_v.at[b]

            def scale_group(g, _):
                vv = vrow[pl.ds(g * L, L)]
                for jj in range(L):
                    v = vv[jj]
                    row = xbuf.at[g * L + jj]
                    for l in range(HB // L):
                        sl = pl.ds(l * L, L)
                        row[sl] = row[sl] * v
                return 0
            lax.fori_loop(0, K // L, scale_group, 0)

        start_g(0, 0)
        start_g(1, 1)
        n_grp = n_batch // NB

        def grp_body(i, _):
            for j in range(NB):
                b = NB * i + j
                wait_g(b, j)
                scale(b, j)
                start_s(b, j)
                qj = (j + 2) % NB
                if j == 0:
                    @pl.when(i >= 1)
                    def _():
                        wait_s(b - 1, qj)
                else:
                    wait_s(b - 1, qj)

                @pl.when(b + 2 < n_batch)
                def _():
                    start_g(b + 2, qj)
            return 0
        lax.fori_loop(0, n_grp, grp_body, 0)
        wait_s(n_batch - 1, (n_batch - 1) % NB)
        plsc.subcore_barrier()

        pltpu.sync_copy(ysh.at[pl.ds(sid * rpt, rpt)],
                        out_h.at[cid, pl.ds(sid * rpt, rpt)])

    return k(xT2, rows3, cols3, vals3, bias)


def kernel(x, rows, cols, values, bias):
    batch, n_in = x.shape
    n_out = bias.shape[0]
    nnz = rows.shape[0]

    grain = NS * K * NB
    nnz_p = ((nnz + grain - 1) // grain) * grain
    pad = nnz_p - nnz
    n_batch = nnz_p // (NS * K)

    xT = x.T
    xT2 = jnp.concatenate([xT[:, :HB], xT[:, HB:]], axis=0)
    rows_p = jnp.concatenate([rows.astype(jnp.int32),
                              jnp.zeros((pad,), jnp.int32)])
    cols_p = jnp.concatenate([cols.astype(jnp.int32),
                              jnp.zeros((pad,), jnp.int32)])
    vals_p = jnp.concatenate([values, jnp.zeros((pad,), jnp.float32)])
    rows3 = rows_p.reshape(NS, n_batch, K)
    cols3 = cols_p.reshape(NS, n_batch, K)
    vals3 = vals_p.reshape(NS, n_batch, K)

    halves = _sc_spmm(xT2, rows3, cols3, vals3, bias, n_out, n_batch)
    y_t = jnp.concatenate([halves[0], halves[1]], axis=1)
    return y_t.T


# speedup vs baseline: 2.1470x; 2.1470x over previous
import functools

import jax
import jax.numpy as jnp
from jax import lax
from jax.experimental import pallas as pl
from jax.experimental.pallas import tpu as pltpu
from jax.experimental.pallas import tpu_sc as plsc

NC = 2
NS = 16
L = 16
K = 128
HB = 128


def _sc_spmm(xT2, rows3, cols3, vals3, bias, n_out, n_batch):
    rpt = n_out // NS
    mesh = plsc.VectorSubcoreMesh(
        core_axis_name="c", subcore_axis_name="s", num_cores=NC,
        num_subcores=NS)

    @functools.partial(
        pl.kernel,
        out_type=jax.ShapeDtypeStruct((NC, n_out, HB), jnp.float32),
        mesh=mesh,
        scratch_types=[
            pltpu.VMEM((K, HB), jnp.float32),
            pltpu.VMEM((K, HB), jnp.float32),
            pltpu.VMEM((n_batch, K), jnp.int32),
            pltpu.VMEM((n_batch, K), jnp.int32),
            pltpu.VMEM((n_batch, K), jnp.float32),
            pltpu.VMEM((rpt,), jnp.float32),
            pltpu.VMEM_SHARED((n_out, HB), jnp.float32),
            pltpu.SemaphoreType.DMA,
            pltpu.SemaphoreType.DMA,
            pltpu.SemaphoreType.DMA,
        ],
    )
    def k(xT2_h, rows_h, cols_h, vals_h, bias_h, out_h,
          xb0, xb1, rows_v, cols_v, vals_v, bias_v, ysh,
          sem0, sem1, sem2):
        cid = lax.axis_index("c")
        sid = lax.axis_index("s")
        xbufs = (xb0, xb1)
        sems = (sem0, sem1)

        pltpu.sync_copy(rows_h.at[sid], rows_v)
        pltpu.sync_copy(cols_h.at[sid], cols_v)
        pltpu.sync_copy(vals_h.at[sid], vals_v)
        coff = jnp.zeros((L,), jnp.int32) + cid * (xT2.shape[0] // NC)

        def shift_row(b, _):
            crow = cols_v.at[b]
            for l in range(K // L):
                sl = pl.ds(l * L, L)
                crow[sl] = crow[sl] + coff
            return 0
        lax.fori_loop(0, n_batch, shift_row, 0)

        pltpu.sync_copy(bias_h.at[pl.ds(sid * rpt, rpt)], bias_v)
        for h in range(rpt // K):
            def init_group(g, _):
                bv = bias_v[pl.ds(h * K + g * L, L)]
                for jj in range(L):
                    rowvec = jnp.zeros((L,), jnp.float32) + bv[jj]
                    row = xb0.at[g * L + jj]
                    for l in range(HB // L):
                        row[pl.ds(l * L, L)] = rowvec
                return 0
            lax.fori_loop(0, K // L, init_group, 0)
            pltpu.sync_copy(xb0, ysh.at[pl.ds(sid * rpt + h * K, K)])
        plsc.subcore_barrier()

        def start(b, p):
            pltpu.async_copy(xT2_h.at[cols_v.at[b]], xbufs[p], sems[p])

        def wait(b, p):
            pltpu.make_async_copy(
                xT2_h.at[cols_v.at[b]], xbufs[p], sems[p]).wait()

        def process(b, p):
            xbuf = xbufs[p]
            vrow = vals_v.at[b]

            def scale_group(g, _):
                vv = vrow[pl.ds(g * L, L)]
                for jj in range(L):
                    v = vv[jj]
                    row = xbuf.at[g * L + jj]
                    for l in range(HB // L):
                        sl = pl.ds(l * L, L)
                        row[sl] = row[sl] * v
                return 0
            lax.fori_loop(0, K // L, scale_group, 0)
            pltpu.async_copy(xbuf, ysh.at[rows_v.at[b]], sem2,
                             add=True)
            pltpu.make_async_copy(
                xbuf, ysh.at[rows_v.at[b]], sem2).wait()

        start(0, 0)

        def pair_body(i, _):
            b0 = 2 * i
            b1 = 2 * i + 1
            start(b1, 1)
            wait(b0, 0)
            process(b0, 0)

            @pl.when(b1 + 1 < n_batch)
            def _():
                start(b1 + 1, 0)
            wait(b1, 1)
            process(b1, 1)
            return 0
        lax.fori_loop(0, n_batch // 2, pair_body, 0)
        plsc.subcore_barrier()

        pltpu.sync_copy(ysh.at[pl.ds(sid * rpt, rpt)],
                        out_h.at[cid, pl.ds(sid * rpt, rpt)])

    return k(xT2, rows3, cols3, vals3, bias)


def kernel(x, rows, cols, values, bias):
    batch, n_in = x.shape
    n_out = bias.shape[0]
    nnz = rows.shape[0]

    grain = NS * K * 2
    nnz_p = ((nnz + grain - 1) // grain) * grain
    pad = nnz_p - nnz
    n_batch = nnz_p // (NS * K)

    xT = x.T
    xT2 = jnp.concatenate([xT[:, :HB], xT[:, HB:]], axis=0)
    rows_p = jnp.concatenate([rows.astype(jnp.int32),
                              jnp.zeros((pad,), jnp.int32)])
    cols_p = jnp.concatenate([cols.astype(jnp.int32),
                              jnp.zeros((pad,), jnp.int32)])
    vals_p = jnp.concatenate([values, jnp.zeros((pad,), jnp.float32)])
    rows3 = rows_p.reshape(NS, n_batch, K)
    cols3 = cols_p.reshape(NS, n_batch, K)
    vals3 = vals_p.reshape(NS, n_batch, K)

    halves = _sc_spmm(xT2, rows3, cols3, vals3, bias, n_out, n_batch)
    y_t = jnp.concatenate([halves[0], halves[1]], axis=1)
    return y_t.T
